# baseline (speedup 1.0000x reference)
import functools

import jax
import jax.numpy as jnp
from jax.experimental import pallas as pl
from jax.experimental.pallas import tpu as pltpu

_K = 100
_BM = 256

_H = -2147450880
_ONE2 = 65537


def _fold_count(m):
    s = m
    while s.shape[1] > 128:
        h = s.shape[1] // 2
        s = s[:, :h] + s[:, h:]
    sc = (jax.lax.shift_right_logical(s, 16) + jnp.bitwise_and(s, 0xFFFF)).astype(
        jnp.float32
    )
    return jnp.sum(sc, axis=1, keepdims=True)


def _body(x_ref, we_ref, be_ref, wd_ref, bd_ref, o_ref):
    xm = (x_ref[...] - bd_ref[...]).astype(jnp.bfloat16)
    pre = jnp.dot(xm, we_ref[...], preferred_element_type=jnp.float32)
    y = jnp.maximum(pre + be_ref[...], 0.0)
    bits = jax.lax.bitcast_convert_type(y, jnp.int32)
    bm = y.shape[0]
    half = bits.shape[1] // 2
    kf = jnp.float32(_K)

    ba = bits[:, :half]
    bb = bits[:, half:]
    ph = jnp.bitwise_or(
        jnp.bitwise_or(
            jnp.bitwise_and(ba, jnp.int32(-65536)),
            jax.lax.shift_right_logical(bb, 16),
        ),
        _H,
    )
    pl15 = jnp.bitwise_or(
        jnp.bitwise_or(
            jnp.bitwise_and(jax.lax.shift_left(ba, 15), jnp.int32(0x7FFF0000)),
            jax.lax.shift_right_logical(jnp.bitwise_and(bb, 0xFFFF), 1),
        ),
        _H,
    )

    def step1(i, t):
        cand = jnp.bitwise_or(t, jax.lax.shift_left(1, 14 - i))
        c2 = jnp.bitwise_or(jax.lax.shift_left(cand, 16), cand)
        d = ph - c2
        m = jnp.bitwise_and(jax.lax.shift_right_arithmetic(d, 15), _ONE2)
        return jnp.where(_fold_count(m) >= kf, cand, t)

    t1 = jax.lax.fori_loop(0, 15, step1, jnp.zeros((bm, 1), jnp.int32))

    cg = t1 + 1
    dg = ph - jnp.bitwise_or(jax.lax.shift_left(cg, 16), cg)
    n_gt = _fold_count(jnp.bitwise_and(jax.lax.shift_right_arithmetic(dg, 15), _ONE2))

    w = jnp.bitwise_xor(_H, jnp.bitwise_or(jax.lax.shift_left(t1, 16), t1))
    z = jnp.bitwise_xor(ph, w)
    uz = jnp.bitwise_or(z, _H) - _ONE2
    meq = jnp.bitwise_and(
        jax.lax.shift_right_arithmetic(jnp.bitwise_and(jnp.bitwise_not(uz), _H), 15),
        _ONE2,
    )

    def step2(i, t):
        cand = jnp.bitwise_or(t, jax.lax.shift_left(1, 14 - i))
        c2 = jnp.bitwise_or(jax.lax.shift_left(cand, 16), cand)
        d = pl15 - c2
        m = jnp.bitwise_and(jax.lax.shift_right_arithmetic(d, 15), meq)
        return jnp.where(n_gt + _fold_count(m) >= kf, cand, t)

    c15 = jax.lax.fori_loop(0, 15, step2, jnp.zeros((bm, 1), jnp.int32))

    thr31 = jnp.bitwise_or(jax.lax.shift_left(t1, 16), jax.lax.shift_left(c15, 1))
    cand_full = jnp.bitwise_or(thr31, 1)
    cnt_full = jnp.sum(
        (bits >= cand_full).astype(jnp.float32), axis=1, keepdims=True
    )
    thr = jnp.where(cnt_full >= kf, cand_full, thr31)

    enc = jnp.where(bits >= thr, y, 0.0).astype(jnp.bfloat16)
    o_ref[...] = (
        jnp.dot(enc, wd_ref[...], preferred_element_type=jnp.float32) + bd_ref[...]
    )


@jax.jit
def kernel(x, W_enc, b_enc, W_dec, b_dec):
    B, d_in = x.shape
    d_sae = W_enc.shape[1]
    be = b_enc.reshape(1, d_sae)
    bd = b_dec.reshape(1, d_in)
    grid = (B // _BM,)
    return pl.pallas_call(
        _body,
        grid=grid,
        in_specs=[
            pl.BlockSpec((_BM, d_in), lambda i: (i, 0)),
            pl.BlockSpec((d_in, d_sae), lambda i: (0, 0)),
            pl.BlockSpec((1, d_sae), lambda i: (0, 0)),
            pl.BlockSpec((d_sae, d_in), lambda i: (0, 0)),
            pl.BlockSpec((1, d_in), lambda i: (0, 0)),
        ],
        out_specs=pl.BlockSpec((_BM, d_in), lambda i: (i, 0)),
        out_shape=jax.ShapeDtypeStruct((B, d_in), jnp.float32),
    )(x, W_enc.astype(jnp.bfloat16), be, W_dec.astype(jnp.bfloat16), bd)


# speedup vs baseline: 1.0032x; 1.0032x over previous
import functools

import jax
import jax.numpy as jnp
from jax.experimental import pallas as pl
from jax.experimental.pallas import tpu as pltpu

_K = 100
_BM = 256

_H = -2147450880
_ONE2 = 65537


def _fold_count(m):
    s = m
    while s.shape[1] > 128:
        h = s.shape[1] // 2
        s = s[:, :h] + s[:, h:]
    sc = (jax.lax.shift_right_logical(s, 16) + jnp.bitwise_and(s, 0xFFFF)).astype(
        jnp.float32
    )
    return jnp.sum(sc, axis=1, keepdims=True)


_G = 4


def _body(x_ref, we_ref, be_ref, wd_ref, bd_ref, o_ref):
    xm = (x_ref[...] - bd_ref[...]).astype(jnp.bfloat16)
    pre = jnp.dot(xm, we_ref[...], preferred_element_type=jnp.float32)
    y = jnp.maximum(pre + be_ref[...], 0.0)
    bits = jax.lax.bitcast_convert_type(y, jnp.int32)
    bm = y.shape[0]
    half = bits.shape[1] // 2
    kf = jnp.float32(_K)
    gm = bm // _G

    ba = bits[:, :half]
    bb = bits[:, half:]
    ph = jnp.bitwise_or(
        jnp.bitwise_or(
            jnp.bitwise_and(ba, jnp.int32(-65536)),
            jax.lax.shift_right_logical(bb, 16),
        ),
        _H,
    )
    pl15 = jnp.bitwise_or(
        jnp.bitwise_or(
            jnp.bitwise_and(jax.lax.shift_left(ba, 15), jnp.int32(0x7FFF0000)),
            jax.lax.shift_right_logical(jnp.bitwise_and(bb, 0xFFFF), 1),
        ),
        _H,
    )
    phs = [ph[g * gm : (g + 1) * gm] for g in range(_G)]
    pls = [pl15[g * gm : (g + 1) * gm] for g in range(_G)]

    def step1(i, ts):
        out = []
        for g in range(_G):
            cand = jnp.bitwise_or(ts[g], jax.lax.shift_left(1, 14 - i))
            c2 = jnp.bitwise_or(jax.lax.shift_left(cand, 16), cand)
            d = phs[g] - c2
            m = jnp.bitwise_and(jax.lax.shift_right_arithmetic(d, 15), _ONE2)
            out.append(jnp.where(_fold_count(m) >= kf, cand, ts[g]))
        return tuple(out)

    z0 = tuple(jnp.zeros((gm, 1), jnp.int32) for _ in range(_G))
    t1s = jax.lax.fori_loop(0, 15, step1, z0)

    n_gts, meqs = [], []
    for g in range(_G):
        cg = t1s[g] + 1
        dg = phs[g] - jnp.bitwise_or(jax.lax.shift_left(cg, 16), cg)
        n_gts.append(
            _fold_count(
                jnp.bitwise_and(jax.lax.shift_right_arithmetic(dg, 15), _ONE2)
            )
        )
        w = jnp.bitwise_xor(
            _H, jnp.bitwise_or(jax.lax.shift_left(t1s[g], 16), t1s[g])
        )
        z = jnp.bitwise_xor(phs[g], w)
        uz = jnp.bitwise_or(z, _H) - _ONE2
        meqs.append(
            jnp.bitwise_and(
                jax.lax.shift_right_arithmetic(
                    jnp.bitwise_and(jnp.bitwise_not(uz), _H), 15
                ),
                _ONE2,
            )
        )

    def step2(i, ts):
        out = []
        for g in range(_G):
            cand = jnp.bitwise_or(ts[g], jax.lax.shift_left(1, 14 - i))
            c2 = jnp.bitwise_or(jax.lax.shift_left(cand, 16), cand)
            d = pls[g] - c2
            m = jnp.bitwise_and(jax.lax.shift_right_arithmetic(d, 15), meqs[g])
            out.append(jnp.where(n_gts[g] + _fold_count(m) >= kf, cand, ts[g]))
        return tuple(out)

    c15s = jax.lax.fori_loop(0, 15, step2, z0)

    t1 = jnp.concatenate(t1s, axis=0)
    c15 = jnp.concatenate(c15s, axis=0)

    thr31 = jnp.bitwise_or(jax.lax.shift_left(t1, 16), jax.lax.shift_left(c15, 1))
    cand_full = jnp.bitwise_or(thr31, 1)
    cnt_full = jnp.sum(
        (bits >= cand_full).astype(jnp.float32), axis=1, keepdims=True
    )
    thr = jnp.where(cnt_full >= kf, cand_full, thr31)

    enc = jnp.where(bits >= thr, y, 0.0).astype(jnp.bfloat16)
    o_ref[...] = (
        jnp.dot(enc, wd_ref[...], preferred_element_type=jnp.float32) + bd_ref[...]
    )


@jax.jit
def kernel(x, W_enc, b_enc, W_dec, b_dec):
    B, d_in = x.shape
    d_sae = W_enc.shape[1]
    be = b_enc.reshape(1, d_sae)
    bd = b_dec.reshape(1, d_in)
    grid = (B // _BM,)
    return pl.pallas_call(
        _body,
        grid=grid,
        in_specs=[
            pl.BlockSpec((_BM, d_in), lambda i: (i, 0)),
            pl.BlockSpec((d_in, d_sae), lambda i: (0, 0)),
            pl.BlockSpec((1, d_sae), lambda i: (0, 0)),
            pl.BlockSpec((d_sae, d_in), lambda i: (0, 0)),
            pl.BlockSpec((1, d_in), lambda i: (0, 0)),
        ],
        out_specs=pl.BlockSpec((_BM, d_in), lambda i: (i, 0)),
        out_shape=jax.ShapeDtypeStruct((B, d_in), jnp.float32),
    )(x, W_enc.astype(jnp.bfloat16), be, W_dec.astype(jnp.bfloat16), bd)


# speedup vs baseline: 1.1203x; 1.1167x over previous
import functools

import jax
import jax.numpy as jnp
from jax.experimental import pallas as pl
from jax.experimental.pallas import tpu as pltpu

_K = 100
_BM = 512


def _tree_count(m_bool):
    s = jnp.where(m_bool, jnp.bfloat16(1), jnp.bfloat16(0))
    while s.shape[1] > 128:
        h = s.shape[1] // 2
        s = s[:, :h] + s[:, h:]
    return jnp.sum(s.astype(jnp.float32), axis=1, keepdims=True)


def _body(x_ref, we_ref, be_ref, wd_ref, bd_ref, o_ref):
    xm = (x_ref[...] - bd_ref[...]).astype(jnp.bfloat16)
    pre = jnp.dot(xm, we_ref[...], preferred_element_type=jnp.float32)
    y = jnp.maximum(pre + be_ref[...], 0.0)
    bits = jax.lax.bitcast_convert_type(y, jnp.int32)
    bm = y.shape[0]
    kf = jnp.float32(_K)

    y16 = jax.lax.bitcast_convert_type(
        jnp.bitwise_and(bits, jnp.int32(-65536)), jnp.float32
    ).astype(jnp.bfloat16)
    lo = (jnp.bitwise_xor(bits, 0x8000) & 0xFFFF).astype(jnp.int16)

    def step1(i, t):
        cand = jnp.bitwise_or(t, jax.lax.shift_left(1, 14 - i))
        cand_b = jax.lax.bitcast_convert_type(cand.astype(jnp.int16), jnp.bfloat16)
        cnt = _tree_count(y16 >= cand_b)
        return jnp.where(cnt >= kf, cand, t)

    t1 = jax.lax.fori_loop(0, 15, step1, jnp.zeros((bm, 1), jnp.int32))
    t1_b = jax.lax.bitcast_convert_type(t1.astype(jnp.int16), jnp.bfloat16)
    n_gt = _tree_count(y16 > t1_b)
    meq = y16 == t1_b

    def step2(i, t):
        cand = jnp.bitwise_or(t, jax.lax.shift_left(1, 15 - i))
        cand16 = jnp.bitwise_xor(cand, 0x8000).astype(jnp.int16)
        cnt = n_gt + _tree_count((lo >= cand16) & meq)
        return jnp.where(cnt >= kf, cand, t)

    u = jax.lax.fori_loop(0, 16, step2, jnp.zeros((bm, 1), jnp.int32))
    thr = jnp.bitwise_or(jax.lax.shift_left(t1, 16), u)

    enc = jnp.where(bits >= thr, y, 0.0).astype(jnp.bfloat16)
    o_ref[...] = (
        jnp.dot(enc, wd_ref[...], preferred_element_type=jnp.float32) + bd_ref[...]
    )


@jax.jit
def kernel(x, W_enc, b_enc, W_dec, b_dec):
    B, d_in = x.shape
    d_sae = W_enc.shape[1]
    be = b_enc.reshape(1, d_sae)
    bd = b_dec.reshape(1, d_in)
    grid = (B // _BM,)
    return pl.pallas_call(
        _body,
        grid=grid,
        in_specs=[
            pl.BlockSpec((_BM, d_in), lambda i: (i, 0)),
            pl.BlockSpec((d_in, d_sae), lambda i: (0, 0)),
            pl.BlockSpec((1, d_sae), lambda i: (0, 0)),
            pl.BlockSpec((d_sae, d_in), lambda i: (0, 0)),
            pl.BlockSpec((1, d_in), lambda i: (0, 0)),
        ],
        out_specs=pl.BlockSpec((_BM, d_in), lambda i: (i, 0)),
        out_shape=jax.ShapeDtypeStruct((B, d_in), jnp.float32),
    )(x, W_enc.astype(jnp.bfloat16), be, W_dec.astype(jnp.bfloat16), bd)
